# baseline (speedup 1.0000x reference)
import jax
import jax.numpy as jnp
from jax import lax
from jax.experimental import pallas as pl
from jax.experimental.pallas import tpu as pltpu


def _make_body(H, W, Bh, R, KP, T, C, NT):
    HW = H * W
    NJ = T // NT

    taps_meta = []
    for kh in range(3):
        for kw in range(3):
            dh, dw = kh - 1, kw - 1
            delta = dh * W + dw
            taps_meta.append(((-delta) % HW, dh, dw))

    def body(cc_s, ab_r, ll_any, dc_any, y5_any, nz_r, out_r,
             taps_s, mask_s, stage_s, dma_sem):
        c = pl.program_id(0)
        j = pl.program_id(1)

        def build_taps(src, store_center):
            for k, (shift, dh, dw) in enumerate(taps_meta):
                if dh == 0 and dw == 0:
                    if store_center:
                        taps_s[k * R:(k + 1) * R, :] = src
                    continue
                tap = pltpu.roll(src, shift=shift, axis=1)
                tap = tap * mask_s[k:k + 1, :]
                taps_s[k * R:(k + 1) * R, :] = tap

        def fetch(src_any):
            cp = pltpu.make_async_copy(
                src_any.at[pl.ds(c * Bh, Bh)], stage_s, dma_sem)
            cp.start()
            cp.wait()
            return stage_s[...].reshape(R, HW)

        @pl.when(j == 0)
        def _init():
            hw_idx = lax.broadcasted_iota(jnp.int32, (1, HW), 1)
            h_pos = hw_idx // W
            w_pos = hw_idx % W
            for k, (_, dh, dw) in enumerate(taps_meta):
                valid = jnp.ones((1, HW), jnp.bool_)
                if dh == -1:
                    valid = jnp.logical_and(valid, h_pos >= 1)
                elif dh == 1:
                    valid = jnp.logical_and(valid, h_pos <= H - 2)
                if dw == -1:
                    valid = jnp.logical_and(valid, w_pos >= 1)
                elif dw == 1:
                    valid = jnp.logical_and(valid, w_pos <= W - 2)
                mask_s[k:k + 1, :] = valid.astype(jnp.float32)

            build_taps(fetch(ll_any), store_center=True)
            e1 = jnp.dot(ab_r[0, :, KP:KP + 9 * R], taps_s[0:9 * R, :],
                         preferred_element_type=jnp.float32)
            build_taps(fetch(dc_any), store_center=True)
            e2 = jnp.dot(ab_r[0, :, KP + 9 * R:KP + 18 * R],
                         taps_s[0:9 * R, :],
                         preferred_element_type=jnp.float32)
            taps_s[9 * R:10 * R, :] = (e1 + e2
                                       + ab_r[1, :, KP + 18 * R:KP + 18 * R + 1])
            taps_s[4 * R:5 * R, :] = fetch(y5_any)

        rc = lax.broadcasted_iota(jnp.int32, (R, 1), 0) % C
        for jj in range(NT):
            t = j * NT + jj
            y = taps_s[4 * R:5 * R, :]
            build_taps(y, store_center=False)
            eps0 = jnp.dot(ab_r[0, :, 0:KP], taps_s[...],
                           preferred_element_type=jnp.float32)

            c1 = cc_s[0, t]
            c2 = cc_s[1, t]
            sv = cc_s[2, t]
            tv = jnp.where(rc == 0, cc_s[3, t],
                           jnp.where(rc == 1, cc_s[4, t], cc_s[5, t]))
            nz = nz_r[:, jj].reshape(R, HW)
            y_new = c1 * y - c2 * eps0 - tv + sv * nz

            if jj == NT - 1:
                @pl.when(j == NJ - 1)
                def _finish():
                    out_r[...] = jnp.clip(y_new, -1.0, 1.0).reshape(
                        Bh, C, H, W)

            taps_s[4 * R:5 * R, :] = y_new

    return body


def _kron_taps(w_ock, Bh):
    C_out, _, C_in = w_ock.shape
    eyeB = jnp.eye(Bh, dtype=jnp.float32)
    big = (eyeB[:, None, None, :, None]
           * w_ock[None, :, :, None, :])
    return big.reshape(Bh * C_out, 9 * Bh * C_in)


def kernel(coef, temb_rev_p, w_lldc, w9p, bias, bw,
           lowlight, data_concate, brightness, y_init, noise_rev):
    B, C, H, W = lowlight.shape
    T = noise_rev.shape[1]
    HW = H * W
    Cp = w9p.shape[0]
    NC = 2
    Bh = B // NC
    R = Bh * C
    KP = 10 * R
    NT = max(d for d in (8, 4, 2, 1) if T % d == 0)
    AW = KP + 18 * R + 128

    w_y = w9p.reshape(Cp, 9, Cp)[:C, :, :C]
    A0 = jnp.concatenate(
        [_kron_taps(w_y, Bh), jnp.eye(R, dtype=jnp.float32)], axis=1)
    w_c = jnp.transpose(w_lldc, (3, 0, 1, 2)).reshape(C, 9, 2 * C)
    A_ll = _kron_taps(w_c[:, :, :C], Bh)
    A_dc = _kron_taps(w_c[:, :, C:], Bh)
    addv = (bias[None, :] + brightness.astype(jnp.float32)[:, None]
            * bw[None, :]).reshape(NC, R, 1)
    ab = jnp.zeros((2, R, AW), jnp.float32)
    ab = ab.at[0, :, 0:KP].set(A0)
    ab = ab.at[0, :, KP:KP + 9 * R].set(A_ll)
    ab = ab.at[0, :, KP + 9 * R:KP + 18 * R].set(A_dc)
    ab = jnp.broadcast_to(ab[None], (NC, 2, R, AW))
    ab = ab.at[:, 1, :, KP + 18 * R:KP + 18 * R + 1].set(addv)
    cc = jnp.concatenate(
        [coef, (coef[1][:, None] * temb_rev_p[:, :C, 0]).T], axis=0)

    out = pl.pallas_call(
        _make_body(H, W, Bh, R, KP, T, C, NT),
        out_shape=jax.ShapeDtypeStruct((B, C, H, W), jnp.float32),
        grid=(NC, T // NT),
        in_specs=[
            pl.BlockSpec(memory_space=pltpu.MemorySpace.SMEM),
            pl.BlockSpec((None, 2, R, AW), lambda c, t: (c, 0, 0, 0)),
            pl.BlockSpec(memory_space=pltpu.MemorySpace.HBM),
            pl.BlockSpec(memory_space=pltpu.MemorySpace.HBM),
            pl.BlockSpec(memory_space=pltpu.MemorySpace.HBM),
            pl.BlockSpec((Bh, NT, C, H, W),
                         lambda c, t: (c, t, 0, 0, 0)),
        ],
        out_specs=pl.BlockSpec((Bh, C, H, W), lambda c, t: (c, 0, 0, 0)),
        scratch_shapes=[
            pltpu.VMEM((KP, HW), jnp.float32),
            pltpu.VMEM((9, HW), jnp.float32),
            pltpu.VMEM((Bh, C, H, W), jnp.float32),
            pltpu.SemaphoreType.DMA,
        ],
        compiler_params=pltpu.CompilerParams(
            dimension_semantics=("parallel", "arbitrary")),
    )(cc, ab, lowlight.astype(jnp.float32),
      data_concate.astype(jnp.float32), y_init.astype(jnp.float32),
      noise_rev.astype(jnp.float32))

    return out


# speedup vs baseline: 1.0404x; 1.0404x over previous
import jax
import jax.numpy as jnp
from jax import lax
from jax.experimental import pallas as pl
from jax.experimental.pallas import tpu as pltpu


def _make_body(H, W, Bh, R, KP, T, C, NT):
    HW = H * W
    NJ = T // NT

    taps_meta = []
    for kh in range(3):
        for kw in range(3):
            dh, dw = kh - 1, kw - 1
            delta = dh * W + dw
            taps_meta.append(((-delta) % HW, dh, dw))

    def body(cc_s, a0_r, ab_r, ll_any, dc_any, y5_any, nz_r, out_r,
             taps_s, mask_s, stage_s, dma_sem):
        c = pl.program_id(0)
        j = pl.program_id(1)

        def build_taps(src, store_center):
            for k, (shift, dh, dw) in enumerate(taps_meta):
                if dh == 0 and dw == 0:
                    if store_center:
                        taps_s[k * R:(k + 1) * R, :] = src
                    continue
                tap = pltpu.roll(src, shift=shift, axis=1)
                tap = tap * mask_s[k:k + 1, :]
                taps_s[k * R:(k + 1) * R, :] = tap

        def fetch(src_any):
            cp = pltpu.make_async_copy(
                src_any.at[pl.ds(c * Bh, Bh)], stage_s, dma_sem)
            cp.start()
            cp.wait()
            return stage_s[...].reshape(R, HW)

        @pl.when(j == 0)
        def _init():
            hw_idx = lax.broadcasted_iota(jnp.int32, (1, HW), 1)
            h_pos = hw_idx // W
            w_pos = hw_idx % W
            for k, (_, dh, dw) in enumerate(taps_meta):
                valid = jnp.ones((1, HW), jnp.bool_)
                if dh == -1:
                    valid = jnp.logical_and(valid, h_pos >= 1)
                elif dh == 1:
                    valid = jnp.logical_and(valid, h_pos <= H - 2)
                if dw == -1:
                    valid = jnp.logical_and(valid, w_pos >= 1)
                elif dw == 1:
                    valid = jnp.logical_and(valid, w_pos <= W - 2)
                mask_s[k:k + 1, :] = valid.astype(jnp.float32)

            build_taps(fetch(ll_any), store_center=True)
            e1 = jnp.dot(ab_r[0, :, 0:9 * R], taps_s[0:9 * R, :],
                         preferred_element_type=jnp.float32)
            build_taps(fetch(dc_any), store_center=True)
            e2 = jnp.dot(ab_r[0, :, 9 * R:18 * R],
                         taps_s[0:9 * R, :],
                         preferred_element_type=jnp.float32)
            taps_s[9 * R:10 * R, :] = (e1 + e2
                                       + ab_r[1, :, 18 * R:18 * R + 1])
            taps_s[4 * R:5 * R, :] = fetch(y5_any)

        rc = lax.broadcasted_iota(jnp.int32, (R, 1), 0) % C
        for jj in range(NT):
            t = j * NT + jj
            y = taps_s[4 * R:5 * R, :]
            build_taps(y, store_center=False)
            eps0 = jnp.dot(a0_r[...], taps_s[...],
                           preferred_element_type=jnp.float32)

            c1 = cc_s[0, t]
            c2 = cc_s[1, t]
            sv = cc_s[2, t]
            tv = jnp.where(rc == 0, cc_s[3, t],
                           jnp.where(rc == 1, cc_s[4, t], cc_s[5, t]))
            nz = nz_r[:, jj].reshape(R, HW)
            y_new = c1 * y - c2 * eps0 - tv + sv * nz

            if jj == NT - 1:
                @pl.when(j == NJ - 1)
                def _finish():
                    out_r[...] = jnp.clip(y_new, -1.0, 1.0).reshape(
                        Bh, C, H, W)

            taps_s[4 * R:5 * R, :] = y_new

    return body


def _kron_taps(w_ock, Bh):
    C_out, _, C_in = w_ock.shape
    eyeB = jnp.eye(Bh, dtype=jnp.float32)
    big = (eyeB[:, None, None, :, None]
           * w_ock[None, :, :, None, :])
    return big.reshape(Bh * C_out, 9 * Bh * C_in)


def kernel(coef, temb_rev_p, w_lldc, w9p, bias, bw,
           lowlight, data_concate, brightness, y_init, noise_rev):
    B, C, H, W = lowlight.shape
    T = noise_rev.shape[1]
    HW = H * W
    Cp = w9p.shape[0]
    NC = 2
    Bh = B // NC
    R = Bh * C
    KP = 10 * R
    NT = max(d for d in (8, 4, 2, 1) if T % d == 0)
    AW = 18 * R + 128

    w_y = w9p.reshape(Cp, 9, Cp)[:C, :, :C]
    A0 = jnp.concatenate(
        [_kron_taps(w_y, Bh), jnp.eye(R, dtype=jnp.float32)], axis=1)
    w_c = jnp.transpose(w_lldc, (3, 0, 1, 2)).reshape(C, 9, 2 * C)
    A_ll = _kron_taps(w_c[:, :, :C], Bh)
    A_dc = _kron_taps(w_c[:, :, C:], Bh)
    addv = (bias[None, :] + brightness.astype(jnp.float32)[:, None]
            * bw[None, :]).reshape(NC, R, 1)
    ab = jnp.zeros((2, R, AW), jnp.float32)
    ab = ab.at[0, :, 0:9 * R].set(A_ll)
    ab = ab.at[0, :, 9 * R:18 * R].set(A_dc)
    ab = jnp.broadcast_to(ab[None], (NC, 2, R, AW))
    ab = ab.at[:, 1, :, 18 * R:18 * R + 1].set(addv)
    cc = jnp.concatenate(
        [coef, (coef[1][:, None] * temb_rev_p[:, :C, 0]).T], axis=0)

    out = pl.pallas_call(
        _make_body(H, W, Bh, R, KP, T, C, NT),
        out_shape=jax.ShapeDtypeStruct((B, C, H, W), jnp.float32),
        grid=(NC, T // NT),
        in_specs=[
            pl.BlockSpec(memory_space=pltpu.MemorySpace.SMEM),
            pl.BlockSpec((R, KP), lambda c, t: (0, 0)),
            pl.BlockSpec((None, 2, R, AW), lambda c, t: (c, 0, 0, 0)),
            pl.BlockSpec(memory_space=pltpu.MemorySpace.HBM),
            pl.BlockSpec(memory_space=pltpu.MemorySpace.HBM),
            pl.BlockSpec(memory_space=pltpu.MemorySpace.HBM),
            pl.BlockSpec((Bh, NT, C, H, W),
                         lambda c, t: (c, t, 0, 0, 0)),
        ],
        out_specs=pl.BlockSpec((Bh, C, H, W), lambda c, t: (c, 0, 0, 0)),
        scratch_shapes=[
            pltpu.VMEM((KP, HW), jnp.float32),
            pltpu.VMEM((9, HW), jnp.float32),
            pltpu.VMEM((Bh, C, H, W), jnp.float32),
            pltpu.SemaphoreType.DMA,
        ],
        compiler_params=pltpu.CompilerParams(
            dimension_semantics=("parallel", "arbitrary")),
    )(cc, A0, ab, lowlight.astype(jnp.float32),
      data_concate.astype(jnp.float32), y_init.astype(jnp.float32),
      noise_rev.astype(jnp.float32))

    return out


# speedup vs baseline: 1.1354x; 1.0913x over previous
import jax
import jax.numpy as jnp
from jax import lax
from jax.experimental import pallas as pl
from jax.experimental.pallas import tpu as pltpu


def _make_body(H, W, Bh, R, KP, T, C, NT):
    HW = H * W
    NJ = T // NT

    taps_meta = []
    for kh in range(3):
        for kw in range(3):
            dh, dw = kh - 1, kw - 1
            delta = dh * W + dw
            taps_meta.append(((-delta) % HW, dh, dw))

    def body(coef_s, ct_s, addv_r, ll_r, dc_r, y5_r, nz_r,
             a0_r, all_r, adc_r, out_r, taps_s, mask_s):
        j = pl.program_id(1)

        def build_taps(src, store_center):
            for k, (shift, dh, dw) in enumerate(taps_meta):
                if dh == 0 and dw == 0:
                    if store_center:
                        taps_s[k * R:(k + 1) * R, :] = src
                    continue
                tap = pltpu.roll(src, shift=shift, axis=1)
                tap = tap * mask_s[k:k + 1, :]
                taps_s[k * R:(k + 1) * R, :] = tap

        @pl.when(j == 0)
        def _init():
            hw_idx = lax.broadcasted_iota(jnp.int32, (1, HW), 1)
            h_pos = hw_idx // W
            w_pos = hw_idx % W
            for k, (_, dh, dw) in enumerate(taps_meta):
                valid = jnp.ones((1, HW), jnp.bool_)
                if dh == -1:
                    valid = jnp.logical_and(valid, h_pos >= 1)
                elif dh == 1:
                    valid = jnp.logical_and(valid, h_pos <= H - 2)
                if dw == -1:
                    valid = jnp.logical_and(valid, w_pos >= 1)
                elif dw == 1:
                    valid = jnp.logical_and(valid, w_pos <= W - 2)
                mask_s[k:k + 1, :] = valid.astype(jnp.float32)

            build_taps(ll_r[...].reshape(R, HW), store_center=True)
            e1 = jnp.dot(all_r[...], taps_s[0:9 * R, :],
                         preferred_element_type=jnp.float32)
            build_taps(dc_r[...].reshape(R, HW), store_center=True)
            e2 = jnp.dot(adc_r[...], taps_s[0:9 * R, :],
                         preferred_element_type=jnp.float32)
            taps_s[9 * R:10 * R, :] = e1 + e2 + addv_r[:, :1]
            taps_s[4 * R:5 * R, :] = y5_r[...].reshape(R, HW)

        rc = lax.broadcasted_iota(jnp.int32, (R, 1), 0) % C
        for jj in range(NT):
            t = j * NT + jj
            y = taps_s[4 * R:5 * R, :]
            build_taps(y, store_center=False)
            eps0 = jnp.dot(a0_r[...], taps_s[...],
                           preferred_element_type=jnp.float32)

            c1 = coef_s[0, t]
            c2 = coef_s[1, t]
            sv = coef_s[2, t]
            tv = jnp.where(rc == 0, ct_s[0, t],
                           jnp.where(rc == 1, ct_s[1, t], ct_s[2, t]))
            nz = nz_r[:, jj].reshape(R, HW)
            y_new = c1 * y - c2 * eps0 - tv + sv * nz

            if jj == NT - 1:
                @pl.when(j == NJ - 1)
                def _finish():
                    out_r[...] = jnp.clip(y_new, -1.0, 1.0).reshape(
                        Bh, C, H, W)

            taps_s[4 * R:5 * R, :] = y_new

    return body


def _kron_taps(w_ock, Bh):
    C_out, _, C_in = w_ock.shape
    eyeB = jnp.eye(Bh, dtype=jnp.float32)
    big = (eyeB[:, None, None, :, None]
           * w_ock[None, :, :, None, :])
    return big.reshape(Bh * C_out, 9 * Bh * C_in)


def kernel(coef, temb_rev_p, w_lldc, w9p, bias, bw,
           lowlight, data_concate, brightness, y_init, noise_rev):
    B, C, H, W = lowlight.shape
    T = noise_rev.shape[1]
    HW = H * W
    Cp = w9p.shape[0]
    NC = 2
    Bh = B // NC
    R = Bh * C
    KP = 10 * R
    NT = max(d for d in (8, 4, 2, 1) if T % d == 0)

    w_y = w9p.reshape(Cp, 9, Cp)[:C, :, :C]
    A0 = jnp.concatenate(
        [_kron_taps(w_y, Bh), jnp.eye(R, dtype=jnp.float32)], axis=1)
    w_c = jnp.transpose(w_lldc, (3, 0, 1, 2)).reshape(C, 9, 2 * C)
    A_ll = _kron_taps(w_c[:, :, :C], Bh)
    A_dc = _kron_taps(w_c[:, :, C:], Bh)
    ct = (coef[1][:, None] * temb_rev_p[:, :C, 0]).T
    addv = (bias[None, :] + brightness.astype(jnp.float32)[:, None]
            * bw[None, :]).reshape(NC, R, 1)
    addv = addv + jnp.zeros((NC, R, 128), jnp.float32)

    out = pl.pallas_call(
        _make_body(H, W, Bh, R, KP, T, C, NT),
        out_shape=jax.ShapeDtypeStruct((B, C, H, W), jnp.float32),
        grid=(NC, T // NT),
        in_specs=[
            pl.BlockSpec(memory_space=pltpu.MemorySpace.SMEM),
            pl.BlockSpec(memory_space=pltpu.MemorySpace.SMEM),
            pl.BlockSpec((None, R, 128), lambda c, t: (c, 0, 0)),
            pl.BlockSpec((Bh, C, H, W), lambda c, t: (c, 0, 0, 0)),
            pl.BlockSpec((Bh, C, H, W), lambda c, t: (c, 0, 0, 0)),
            pl.BlockSpec((Bh, C, H, W), lambda c, t: (c, 0, 0, 0)),
            pl.BlockSpec((Bh, NT, C, H, W),
                         lambda c, t: (c, t, 0, 0, 0)),
            pl.BlockSpec((R, KP), lambda c, t: (0, 0)),
            pl.BlockSpec((R, 9 * R), lambda c, t: (0, 0)),
            pl.BlockSpec((R, 9 * R), lambda c, t: (0, 0)),
        ],
        out_specs=pl.BlockSpec((Bh, C, H, W), lambda c, t: (c, 0, 0, 0)),
        scratch_shapes=[
            pltpu.VMEM((KP, HW), jnp.float32),
            pltpu.VMEM((9, HW), jnp.float32),
        ],
        compiler_params=pltpu.CompilerParams(
            dimension_semantics=("parallel", "arbitrary")),
    )(coef, ct, addv, lowlight.astype(jnp.float32),
      data_concate.astype(jnp.float32), y_init.astype(jnp.float32),
      noise_rev.astype(jnp.float32), A0, A_ll, A_dc)

    return out
